# baseline (speedup 1.0000x reference)
import functools

import jax
import jax.numpy as jnp
from jax import lax
from jax.experimental import pallas as pl
from jax.experimental.pallas import tpu as pltpu
from jax.experimental.pallas import tpu_sc as plsc

N = 10000
D = 256
DH = 128
K = 5
EPS = 1e-5
ALPHA = 0.01

L = 16
NC = 2
NS = 16
CHUNK = 128
DUMMY = N
N_PAD = 10112
RPT = N_PAD // NS

E_PAD = 163840
PREP_CHUNKS = E_PAD // (CHUNK * NC * NS)
SPMV_CHUNKS = E_PAD // (CHUNK * NS)
STAGE_CHUNKS = SPMV_CHUNKS // 2

BR = 400
NB = N // BR

_mesh = plsc.VectorSubcoreMesh(core_axis_name="c", subcore_axis_name="s",
                               num_cores=NC, num_subcores=NS)


def _prep_body(row_hbm, col_hbm, rowp_hbm, col2_hbm,
               rall, call, rpall, c1all):
    c = lax.axis_index("c")
    s = lax.axis_index("s")
    t = c * NS + s
    base = pl.multiple_of(t * PREP_CHUNKS, 8)
    pltpu.sync_copy(row_hbm.at[pl.ds(base, PREP_CHUNKS)], rall)
    pltpu.sync_copy(col_hbm.at[pl.ds(base, PREP_CHUNKS)], call)

    def chunk_body(i, carry):
        def vec_body(j, carry2):
            o = pl.multiple_of(j * L, L)
            r = rall[i, pl.ds(o, L)]
            cv = call[i, pl.ds(o, L)]
            rpall[i, pl.ds(o, L)] = jnp.where(r != cv, r, DUMMY)
            c1all[i, pl.ds(o, L)] = cv + N
            return carry2

        lax.fori_loop(0, CHUNK // L, vec_body, 0)
        return carry

    lax.fori_loop(0, PREP_CHUNKS, chunk_body, 0)
    pltpu.sync_copy(rpall, rowp_hbm.at[pl.ds(base, PREP_CHUNKS)])
    pltpu.sync_copy(call, col2_hbm.at[0, pl.ds(base, PREP_CHUNKS)])
    pltpu.sync_copy(c1all, col2_hbm.at[1, pl.ds(base, PREP_CHUNKS)])


_prep = pl.kernel(
    _prep_body,
    name="sc_prep",
    out_type=(
        jax.ShapeDtypeStruct((E_PAD // CHUNK, CHUNK), jnp.int32),
        jax.ShapeDtypeStruct((2, E_PAD // CHUNK, CHUNK), jnp.int32),
    ),
    mesh=_mesh,
    scratch_types=[
        pltpu.VMEM((PREP_CHUNKS, CHUNK), jnp.int32),
        pltpu.VMEM((PREP_CHUNKS, CHUNK), jnp.int32),
        pltpu.VMEM((PREP_CHUNKS, CHUNK), jnp.int32),
        pltpu.VMEM((PREP_CHUNKS, CHUNK), jnp.int32),
    ],
)

def _spmv_body(rowp_hbm, col2_hbm, u_hbm, zeros_hbm, z_hbm,
               cb_all, rb_all, db0, db1, z_sp, sg0, sg1):
    c = lax.axis_index("c")
    s = lax.axis_index("s")
    dbs = (db0, db1)
    sgs = (sg0, sg1)
    pltpu.sync_copy(zeros_hbm.at[pl.ds(pl.multiple_of(s * RPT, 8), RPT)],
                    z_sp.at[pl.ds(pl.multiple_of(s * RPT, 8), RPT)])
    plsc.subcore_barrier()

    def fire_gather(q, b):
        pltpu.async_copy(u_hbm.at[cb_all.at[q]], dbs[b], sgs[b])

    def wait_gather(b):
        pltpu.make_async_copy(u_hbm.at[cb_all.at[0]], dbs[b], sgs[b]).wait()

    nstage = SPMV_CHUNKS // STAGE_CHUNKS
    for stage in range(nstage):
        base = pl.multiple_of(s * SPMV_CHUNKS + stage * STAGE_CHUNKS, 8)
        pltpu.sync_copy(col2_hbm.at[c, pl.ds(base, STAGE_CHUNKS)], cb_all)
        pltpu.sync_copy(rowp_hbm.at[pl.ds(base, STAGE_CHUNKS)], rb_all)
        for b in range(2):
            fire_gather(b, b)

        def group_body(g, carry):
            for b in range(2):
                q = 2 * g + b
                wait_gather(b)
                pltpu.sync_copy(dbs[b], z_sp.at[rb_all.at[q]], add=True)

                @pl.when(q + 2 < STAGE_CHUNKS)
                def _():
                    fire_gather(q + 2, b)
            return carry

        lax.fori_loop(0, STAGE_CHUNKS // 2, group_body, 0)

    plsc.subcore_barrier()
    pltpu.sync_copy(z_sp.at[pl.ds(pl.multiple_of(s * RPT, 8), RPT)],
                    z_hbm.at[c, pl.ds(pl.multiple_of(s * RPT, 8), RPT)])


_spmv = pl.kernel(
    _spmv_body,
    name="sc_spmv",
    out_type=jax.ShapeDtypeStruct((NC, N_PAD, DH), jnp.float32),
    mesh=_mesh,
    scratch_types=[
        pltpu.VMEM((STAGE_CHUNKS, CHUNK), jnp.int32),
        pltpu.VMEM((STAGE_CHUNKS, CHUNK), jnp.int32),
        pltpu.VMEM((CHUNK, DH), jnp.float32),
        pltpu.VMEM((CHUNK, DH), jnp.float32),
        pltpu.VMEM_SHARED((N_PAD, DH), jnp.float32),
        pltpu.SemaphoreType.DMA,
        pltpu.SemaphoreType.DMA,
    ],
)


def _dinv_u_body(deg0_ref, x_ref, dinv_ref, u_ref):
    d = deg0_ref[0, :, 0:1]
    dinv = jnp.where(d > 0.0, lax.rsqrt(jnp.maximum(d, 1e-30)), 0.0)
    dinv_ref[...] = jnp.broadcast_to(dinv, (BR, DH))
    u_ref[...] = dinv * x_ref[...]


def _dinv_u(degp, x):
    return pl.pallas_call(
        _dinv_u_body,
        grid=(NB, 2),
        in_specs=[
            pl.BlockSpec((1, BR, DH), lambda i, h: (0, i, 0)),
            pl.BlockSpec((BR, DH), lambda i, h: (i, h)),
        ],
        out_specs=[
            pl.BlockSpec((BR, DH), lambda i, h: (i, 0)),
            pl.BlockSpec((BR, DH), lambda i, h: (i + h * NB, 0)),
        ],
        out_shape=[
            jax.ShapeDtypeStruct((N, DH), jnp.float32),
            jax.ShapeDtypeStruct((2 * N, DH), jnp.float32),
        ],
    )(degp, x)


def _step_body_first(z_ref, dinv_ref, t_ref, u_ref):
    t = -1.0 * dinv_ref[...] * z_ref[0]
    t_ref[...] = t
    u_ref[...] = dinv_ref[...] * t


def _step_body_mid(z_ref, dinv_ref, tprev2_ref, t_ref, u_ref):
    t = -2.0 * dinv_ref[...] * z_ref[0] - tprev2_ref[...]
    t_ref[...] = t
    u_ref[...] = dinv_ref[...] * t


def _step_body_last(z_ref, dinv_ref, tprev2_ref, t_ref):
    t_ref[...] = -2.0 * dinv_ref[...] * z_ref[0] - tprev2_ref[...]


def _step(z, dinv, tprev2, first, need_u):
    z_spec = pl.BlockSpec((1, BR, DH), lambda i, h: (h, i, 0))
    dinv_spec = pl.BlockSpec((BR, DH), lambda i, h: (i, 0))
    nd_spec = pl.BlockSpec((BR, DH), lambda i, h: (i, h))
    u_spec = pl.BlockSpec((BR, DH), lambda i, h: (i + h * NB, 0))
    t_shape = jax.ShapeDtypeStruct((N, D), jnp.float32)
    u_shape = jax.ShapeDtypeStruct((2 * N, DH), jnp.float32)
    if first:
        return pl.pallas_call(
            _step_body_first, grid=(NB, 2),
            in_specs=[z_spec, dinv_spec],
            out_specs=[nd_spec, u_spec],
            out_shape=[t_shape, u_shape],
        )(z, dinv)
    if need_u:
        return pl.pallas_call(
            _step_body_mid, grid=(NB, 2),
            in_specs=[z_spec, dinv_spec, nd_spec],
            out_specs=[nd_spec, u_spec],
            out_shape=[t_shape, u_shape],
        )(z, dinv, tprev2)
    return pl.pallas_call(
        _step_body_last, grid=(NB, 2),
        in_specs=[z_spec, dinv_spec, nd_spec],
        out_specs=nd_spec,
        out_shape=t_shape,
    )(z, dinv, tprev2)


def _matmul_body(t0_ref, t1_ref, t2_ref, t3_ref, t4_ref, w_ref, b_ref,
                 out_ref, stats_ref):
    acc = jnp.dot(t0_ref[...], w_ref[0], preferred_element_type=jnp.float32)
    acc += jnp.dot(t1_ref[...], w_ref[1], preferred_element_type=jnp.float32)
    acc += jnp.dot(t2_ref[...], w_ref[2], preferred_element_type=jnp.float32)
    acc += jnp.dot(t3_ref[...], w_ref[3], preferred_element_type=jnp.float32)
    acc += jnp.dot(t4_ref[...], w_ref[4], preferred_element_type=jnp.float32)
    out = acc + b_ref[...]
    out_ref[...] = out
    s1 = jnp.sum(out, axis=0, keepdims=True)
    s2 = jnp.sum(out * out, axis=0, keepdims=True)
    part = jnp.concatenate([s1, s2, jnp.zeros((6, D), jnp.float32)], axis=0)

    @pl.when(pl.program_id(0) == 0)
    def _():
        stats_ref[...] = part

    @pl.when(pl.program_id(0) > 0)
    def _():
        stats_ref[...] = stats_ref[...] + part


def _matmul_stats(ts, w, b2):
    nd_spec = pl.BlockSpec((BR, D), lambda i: (i, 0))
    return pl.pallas_call(
        _matmul_body,
        grid=(NB,),
        in_specs=[nd_spec] * 5 + [
            pl.BlockSpec((K, D, D), lambda i: (0, 0, 0)),
            pl.BlockSpec((1, D), lambda i: (0, 0)),
        ],
        out_specs=[nd_spec, pl.BlockSpec((8, D), lambda i: (0, 0))],
        out_shape=[
            jax.ShapeDtypeStruct((N, D), jnp.float32),
            jax.ShapeDtypeStruct((8, D), jnp.float32),
        ],
    )(*ts, w, b2)


def _bn_body(o_ref, stats_ref, gamma_ref, beta_ref, y_ref):
    mean = stats_ref[0:1, :] * (1.0 / N)
    ex2 = stats_ref[1:2, :] * (1.0 / N)
    var = ex2 - mean * mean
    inv = lax.rsqrt(var + EPS)
    y = (o_ref[...] - mean) * inv * gamma_ref[...] + beta_ref[...]
    y_ref[...] = jnp.where(y >= 0.0, y, ALPHA * y)


def _bn(out_lin, stats, gamma2, beta2):
    nd_spec = pl.BlockSpec((BR, D), lambda i: (i, 0))
    return pl.pallas_call(
        _bn_body,
        grid=(NB,),
        in_specs=[
            nd_spec,
            pl.BlockSpec((8, D), lambda i: (0, 0)),
            pl.BlockSpec((1, D), lambda i: (0, 0)),
            pl.BlockSpec((1, D), lambda i: (0, 0)),
        ],
        out_specs=nd_spec,
        out_shape=jax.ShapeDtypeStruct((N, D), jnp.float32),
    )(out_lin, stats, gamma2, beta2)


def kernel(x, edge_idx, W, b, gamma, beta):
    row = edge_idx[0]
    col = edge_idx[1]
    e = row.shape[0]
    pad = E_PAD - e
    row_p = jnp.concatenate([row, jnp.full((pad,), N, jnp.int32)])
    col_p = jnp.concatenate([col, jnp.zeros((pad,), jnp.int32)])
    row_p = row_p.reshape(E_PAD // CHUNK, CHUNK)
    col_p = col_p.reshape(E_PAD // CHUNK, CHUNK)
    zeros_dh = jnp.zeros((N_PAD, DH), jnp.float32)
    ones_tab = jnp.ones((2 * N, DH), jnp.float32)

    rowp, col2 = _prep(row_p, col_p)
    degp = _spmv(rowp, col2, ones_tab, zeros_dh)
    dinv, u_cur = _dinv_u(degp, x)

    ts = [x]
    for k in range(1, K):
        z = _spmv(rowp, col2, u_cur, zeros_dh)
        first = k == 1
        need_u = k < K - 1
        out = _step(z, dinv, None if first else ts[k - 2], first, need_u)
        if need_u:
            t_k, u_cur = out
        else:
            t_k = out
        ts.append(t_k)

    out_lin, stats = _matmul_stats(ts, W, b.reshape(1, D))
    return _bn(out_lin, stats, gamma.reshape(1, D), beta.reshape(1, D))


# speedup vs baseline: 1.1688x; 1.1688x over previous
import functools

import jax
import jax.numpy as jnp
from jax import lax
from jax.experimental import pallas as pl
from jax.experimental.pallas import tpu as pltpu
from jax.experimental.pallas import tpu_sc as plsc

N = 10000
D = 256
DH = 128
K = 5
EPS = 1e-5
ALPHA = 0.01

L = 16
NC = 2
NS = 16
CHUNK = 128
DUMMY = N
N_PAD = 10112
RPT = N_PAD // NS

E_PAD = 163840
PREP_CHUNKS = E_PAD // (CHUNK * NC * NS)
SPMV_CHUNKS = E_PAD // (CHUNK * NS)
STAGE_CHUNKS = SPMV_CHUNKS // 5

BR = 400
NB = N // BR

_mesh = plsc.VectorSubcoreMesh(core_axis_name="c", subcore_axis_name="s",
                               num_cores=NC, num_subcores=NS)


def _prep_body(row_hbm, col_hbm, ones_hbm, zeros_hbm,
               rowp_hbm, col2_hbm, deg_hbm,
               rall, call, rpall, c1all, ones_v, deg_sp, sem_sc):
    c = lax.axis_index("c")
    s = lax.axis_index("s")
    t = c * NS + s
    base = pl.multiple_of(t * PREP_CHUNKS, 8)
    pltpu.sync_copy(ones_hbm, ones_v)
    pltpu.sync_copy(row_hbm.at[pl.ds(base, PREP_CHUNKS)], rall)
    pltpu.sync_copy(col_hbm.at[pl.ds(base, PREP_CHUNKS)], call)
    pltpu.sync_copy(zeros_hbm.at[pl.ds(pl.multiple_of(s * RPT, 8), RPT)],
                    deg_sp.at[pl.ds(pl.multiple_of(s * RPT, 8), RPT)])

    def chunk_body(i, carry):
        def vec_body(j, carry2):
            o = pl.multiple_of(j * L, L)
            r = rall[i, pl.ds(o, L)]
            cv = call[i, pl.ds(o, L)]
            rpall[i, pl.ds(o, L)] = jnp.where(r != cv, r, DUMMY)
            c1all[i, pl.ds(o, L)] = cv + N
            return carry2

        lax.fori_loop(0, CHUNK // L, vec_body, 0)
        return carry

    lax.fori_loop(0, PREP_CHUNKS, chunk_body, 0)
    plsc.subcore_barrier()

    def deg_group(g, carry):
        for b in range(4):
            pltpu.async_copy(ones_v, deg_sp.at[rpall.at[4 * g + b]], sem_sc,
                             add=True)
        for b in range(4):
            pltpu.make_async_copy(ones_v, deg_sp.at[rpall.at[0]], sem_sc).wait()
        return carry

    lax.fori_loop(0, PREP_CHUNKS // 4, deg_group, 0)
    pltpu.sync_copy(rpall, rowp_hbm.at[pl.ds(base, PREP_CHUNKS)])
    pltpu.sync_copy(call, col2_hbm.at[0, pl.ds(base, PREP_CHUNKS)])
    pltpu.sync_copy(c1all, col2_hbm.at[1, pl.ds(base, PREP_CHUNKS)])
    plsc.subcore_barrier()
    pltpu.sync_copy(deg_sp.at[pl.ds(pl.multiple_of(s * RPT, 8), RPT)],
                    deg_hbm.at[c, pl.ds(pl.multiple_of(s * RPT, 8), RPT)])


_prep = pl.kernel(
    _prep_body,
    name="sc_prep",
    out_type=(
        jax.ShapeDtypeStruct((E_PAD // CHUNK, CHUNK), jnp.int32),
        jax.ShapeDtypeStruct((2, E_PAD // CHUNK, CHUNK), jnp.int32),
        jax.ShapeDtypeStruct((NC, N_PAD, DH), jnp.float32),
    ),
    mesh=_mesh,
    scratch_types=[
        pltpu.VMEM((PREP_CHUNKS, CHUNK), jnp.int32),
        pltpu.VMEM((PREP_CHUNKS, CHUNK), jnp.int32),
        pltpu.VMEM((PREP_CHUNKS, CHUNK), jnp.int32),
        pltpu.VMEM((PREP_CHUNKS, CHUNK), jnp.int32),
        pltpu.VMEM((CHUNK, DH), jnp.float32),
        pltpu.VMEM_SHARED((N_PAD, DH), jnp.float32),
        pltpu.SemaphoreType.DMA,
    ],
)

def _spmv_body(rowp_hbm, col2_hbm, u_hbm, zeros_hbm, z_hbm,
               cb_all, rb_all, db0, db1, z_sp, sg0, sg1, ss0, ss1):
    c = lax.axis_index("c")
    s = lax.axis_index("s")
    dbs = (db0, db1)
    sgs = (sg0, sg1)
    sss = (ss0, ss1)
    pltpu.sync_copy(zeros_hbm.at[pl.ds(pl.multiple_of(s * RPT, 8), RPT)],
                    z_sp.at[pl.ds(pl.multiple_of(s * RPT, 8), RPT)])
    plsc.subcore_barrier()

    def fire_gather(q, b):
        pltpu.async_copy(u_hbm.at[cb_all.at[q]], dbs[b], sgs[b])

    def wait_gather(b):
        pltpu.make_async_copy(u_hbm.at[cb_all.at[0]], dbs[b], sgs[b]).wait()

    def fire_scatter(q, b):
        pltpu.async_copy(dbs[b], z_sp.at[rb_all.at[q]], sss[b], add=True)

    def wait_scatter(b):
        pltpu.make_async_copy(dbs[b], z_sp.at[rb_all.at[0]], sss[b]).wait()

    nstage = SPMV_CHUNKS // STAGE_CHUNKS
    for stage in range(nstage):
        base = pl.multiple_of(s * SPMV_CHUNKS + stage * STAGE_CHUNKS, 8)
        pltpu.sync_copy(col2_hbm.at[c, pl.ds(base, STAGE_CHUNKS)], cb_all)
        pltpu.sync_copy(rowp_hbm.at[pl.ds(base, STAGE_CHUNKS)], rb_all)
        for b in range(2):
            fire_gather(b, b)

        def group_body(g, carry):
            for b in range(2):
                q = 2 * g + b
                wait_gather(b)
                fire_scatter(q, b)

                @pl.when(q + 2 < STAGE_CHUNKS)
                def _():
                    wait_scatter(b)
                    fire_gather(q + 2, b)
            return carry

        lax.fori_loop(0, STAGE_CHUNKS // 2, group_body, 0)
        for b in range(2):
            wait_scatter(b)

    plsc.subcore_barrier()
    pltpu.sync_copy(z_sp.at[pl.ds(pl.multiple_of(s * RPT, 8), RPT)],
                    z_hbm.at[c, pl.ds(pl.multiple_of(s * RPT, 8), RPT)])


_spmv = pl.kernel(
    _spmv_body,
    name="sc_spmv",
    out_type=jax.ShapeDtypeStruct((NC, N_PAD, DH), jnp.float32),
    mesh=_mesh,
    scratch_types=[
        pltpu.VMEM((STAGE_CHUNKS, CHUNK), jnp.int32),
        pltpu.VMEM((STAGE_CHUNKS, CHUNK), jnp.int32),
        pltpu.VMEM((CHUNK, DH), jnp.float32),
        pltpu.VMEM((CHUNK, DH), jnp.float32),
        pltpu.VMEM_SHARED((N_PAD, DH), jnp.float32),
        pltpu.SemaphoreType.DMA,
        pltpu.SemaphoreType.DMA,
        pltpu.SemaphoreType.DMA,
        pltpu.SemaphoreType.DMA,
    ],
)


def _dinv_u_body(deg0_ref, deg1_ref, x_ref, dinv_ref, u_ref):
    d = deg0_ref[0, :, 0:1] + deg1_ref[0, :, 0:1]
    dinv = jnp.where(d > 0.0, lax.rsqrt(jnp.maximum(d, 1e-30)), 0.0)
    dinv_ref[...] = jnp.broadcast_to(dinv, (BR, DH))
    u_ref[...] = dinv * x_ref[...]


def _dinv_u(degp, x):
    return pl.pallas_call(
        _dinv_u_body,
        grid=(NB, 2),
        in_specs=[
            pl.BlockSpec((1, BR, DH), lambda i, h: (0, i, 0)),
            pl.BlockSpec((1, BR, DH), lambda i, h: (1, i, 0)),
            pl.BlockSpec((BR, DH), lambda i, h: (i, h)),
        ],
        out_specs=[
            pl.BlockSpec((BR, DH), lambda i, h: (i, 0)),
            pl.BlockSpec((BR, DH), lambda i, h: (i + h * NB, 0)),
        ],
        out_shape=[
            jax.ShapeDtypeStruct((N, DH), jnp.float32),
            jax.ShapeDtypeStruct((2 * N, DH), jnp.float32),
        ],
    )(degp, degp, x)


def _step_body_first(z_ref, dinv_ref, t_ref, u_ref):
    t = -1.0 * dinv_ref[...] * z_ref[0]
    t_ref[...] = t
    u_ref[...] = dinv_ref[...] * t


def _step_body_mid(z_ref, dinv_ref, tprev2_ref, t_ref, u_ref):
    t = -2.0 * dinv_ref[...] * z_ref[0] - tprev2_ref[...]
    t_ref[...] = t
    u_ref[...] = dinv_ref[...] * t


def _step_body_last(z_ref, dinv_ref, tprev2_ref, t_ref):
    t_ref[...] = -2.0 * dinv_ref[...] * z_ref[0] - tprev2_ref[...]


def _step(z, dinv, tprev2, first, need_u):
    z_spec = pl.BlockSpec((1, BR, DH), lambda i, h: (h, i, 0))
    dinv_spec = pl.BlockSpec((BR, DH), lambda i, h: (i, 0))
    nd_spec = pl.BlockSpec((BR, DH), lambda i, h: (i, h))
    u_spec = pl.BlockSpec((BR, DH), lambda i, h: (i + h * NB, 0))
    t_shape = jax.ShapeDtypeStruct((N, D), jnp.float32)
    u_shape = jax.ShapeDtypeStruct((2 * N, DH), jnp.float32)
    if first:
        return pl.pallas_call(
            _step_body_first, grid=(NB, 2),
            in_specs=[z_spec, dinv_spec],
            out_specs=[nd_spec, u_spec],
            out_shape=[t_shape, u_shape],
        )(z, dinv)
    if need_u:
        return pl.pallas_call(
            _step_body_mid, grid=(NB, 2),
            in_specs=[z_spec, dinv_spec, nd_spec],
            out_specs=[nd_spec, u_spec],
            out_shape=[t_shape, u_shape],
        )(z, dinv, tprev2)
    return pl.pallas_call(
        _step_body_last, grid=(NB, 2),
        in_specs=[z_spec, dinv_spec, nd_spec],
        out_specs=nd_spec,
        out_shape=t_shape,
    )(z, dinv, tprev2)


def _matmul_body(t0_ref, t1_ref, t2_ref, t3_ref, t4_ref, w_ref, b_ref,
                 out_ref, stats_ref):
    acc = jnp.dot(t0_ref[...], w_ref[0], preferred_element_type=jnp.float32)
    acc += jnp.dot(t1_ref[...], w_ref[1], preferred_element_type=jnp.float32)
    acc += jnp.dot(t2_ref[...], w_ref[2], preferred_element_type=jnp.float32)
    acc += jnp.dot(t3_ref[...], w_ref[3], preferred_element_type=jnp.float32)
    acc += jnp.dot(t4_ref[...], w_ref[4], preferred_element_type=jnp.float32)
    out = acc + b_ref[...]
    out_ref[...] = out
    s1 = jnp.sum(out, axis=0, keepdims=True)
    s2 = jnp.sum(out * out, axis=0, keepdims=True)
    part = jnp.concatenate([s1, s2, jnp.zeros((6, D), jnp.float32)], axis=0)

    @pl.when(pl.program_id(0) == 0)
    def _():
        stats_ref[...] = part

    @pl.when(pl.program_id(0) > 0)
    def _():
        stats_ref[...] = stats_ref[...] + part


def _matmul_stats(ts, w, b2):
    nd_spec = pl.BlockSpec((BR, D), lambda i: (i, 0))
    return pl.pallas_call(
        _matmul_body,
        grid=(NB,),
        in_specs=[nd_spec] * 5 + [
            pl.BlockSpec((K, D, D), lambda i: (0, 0, 0)),
            pl.BlockSpec((1, D), lambda i: (0, 0)),
        ],
        out_specs=[nd_spec, pl.BlockSpec((8, D), lambda i: (0, 0))],
        out_shape=[
            jax.ShapeDtypeStruct((N, D), jnp.float32),
            jax.ShapeDtypeStruct((8, D), jnp.float32),
        ],
    )(*ts, w, b2)


def _bn_body(o_ref, stats_ref, gamma_ref, beta_ref, y_ref):
    mean = stats_ref[0:1, :] * (1.0 / N)
    ex2 = stats_ref[1:2, :] * (1.0 / N)
    var = ex2 - mean * mean
    inv = lax.rsqrt(var + EPS)
    y = (o_ref[...] - mean) * inv * gamma_ref[...] + beta_ref[...]
    y_ref[...] = jnp.where(y >= 0.0, y, ALPHA * y)


def _bn(out_lin, stats, gamma2, beta2):
    nd_spec = pl.BlockSpec((BR, D), lambda i: (i, 0))
    return pl.pallas_call(
        _bn_body,
        grid=(NB,),
        in_specs=[
            nd_spec,
            pl.BlockSpec((8, D), lambda i: (0, 0)),
            pl.BlockSpec((1, D), lambda i: (0, 0)),
            pl.BlockSpec((1, D), lambda i: (0, 0)),
        ],
        out_specs=nd_spec,
        out_shape=jax.ShapeDtypeStruct((N, D), jnp.float32),
    )(out_lin, stats, gamma2, beta2)


def kernel(x, edge_idx, W, b, gamma, beta):
    row = edge_idx[0]
    col = edge_idx[1]
    e = row.shape[0]
    pad = E_PAD - e
    row_p = jnp.concatenate([row, jnp.full((pad,), N, jnp.int32)])
    col_p = jnp.concatenate([col, jnp.zeros((pad,), jnp.int32)])
    row_p = row_p.reshape(E_PAD // CHUNK, CHUNK)
    col_p = col_p.reshape(E_PAD // CHUNK, CHUNK)
    zeros_dh = jnp.zeros((N_PAD, DH), jnp.float32)
    ones_in = jnp.ones((CHUNK, DH), jnp.float32)

    rowp, col2, degp = _prep(row_p, col_p, ones_in, zeros_dh)
    dinv, u_cur = _dinv_u(degp, x)

    ts = [x]
    for k in range(1, K):
        z = _spmv(rowp, col2, u_cur, zeros_dh)
        first = k == 1
        need_u = k < K - 1
        out = _step(z, dinv, None if first else ts[k - 2], first, need_u)
        if need_u:
            t_k, u_cur = out
        else:
            t_k = out
        ts.append(t_k)

    out_lin, stats = _matmul_stats(ts, W, b.reshape(1, D))
    return _bn(out_lin, stats, gamma.reshape(1, D), beta.reshape(1, D))
